# baseline (speedup 1.0000x reference)
import functools

import jax
import jax.numpy as jnp
from jax import lax
from jax.experimental import pallas as pl
from jax.experimental.pallas import tpu as pltpu
from jax.experimental.pallas import tpu_sc as plsc

B = 16384
V = 100000
D_GMF = 32
D_MLP = 64
NC = 2
NS = 16
NW = NC * NS
ROWS_PER_W = B // NW
CHUNK = 128
CHUNKS_PER_W = ROWS_PER_W // CHUNK
VBLK = 4000
L = 16


def _compact_body(ue_ref, ie_ref, um_ref, im_ref,
                  ue_o, ie_o, um_o, im_o):
    def pack(x, r):
        n = x.shape[0] // r
        x3 = x.reshape(n, r, x.shape[1])
        return jnp.concatenate([x3[:, j, :] for j in range(r)], axis=-1)
    ue_o[...] = pack(ue_ref[...], 4)
    ie_o[...] = pack(ie_ref[...], 4)
    um_o[...] = pack(um_ref[...], 2)
    im_o[...] = pack(im_ref[...], 2)


def _tc_compact(ue, ie, um, im):
    grid = (V // VBLK,)
    bin32 = pl.BlockSpec((VBLK, D_GMF), lambda i: (i, 0))
    bin64 = pl.BlockSpec((VBLK, D_MLP), lambda i: (i, 0))
    bo32 = pl.BlockSpec((VBLK // 4, 128), lambda i: (i, 0))
    bo64 = pl.BlockSpec((VBLK // 2, 128), lambda i: (i, 0))
    return pl.pallas_call(
        _compact_body,
        grid=grid,
        in_specs=[bin32, bin32, bin64, bin64],
        out_specs=[bo32, bo32, bo64, bo64],
        out_shape=[
            jax.ShapeDtypeStruct((V // 4, 128), jnp.float32),
            jax.ShapeDtypeStruct((V // 4, 128), jnp.float32),
            jax.ShapeDtypeStruct((V // 2, 128), jnp.float32),
            jax.ShapeDtypeStruct((V // 2, 128), jnp.float32),
        ],
    )(ue, ie, um, im)


def _gather_side_body(ids_hbm, emb_hbm, mlp_hbm, out,
                      idx, ge_idx, gm_idx, emb_g, mlp_g, pack_v, sem):
    wid = lax.axis_index("s") * NC + lax.axis_index("c")
    base = wid * ROWS_PER_W
    pltpu.sync_copy(ids_hbm.at[pl.ds(wid * CHUNKS_PER_W, CHUNKS_PER_W)], idx)
    for j in range(CHUNKS_PER_W):
        for c in range(CHUNK // L):
            sl = pl.ds(c * L, L)
            v = idx[j, sl]
            ge_idx[j, sl] = v >> 2
            gm_idx[j, sl] = v >> 1
    iota = lax.iota(jnp.int32, L)
    for j in range(CHUNKS_PER_W):
        pltpu.async_copy(emb_hbm.at[ge_idx.at[j]], emb_g, sem).wait()
        pltpu.async_copy(mlp_hbm.at[gm_idx.at[j]], mlp_g, sem).wait()

        def extract(r, _):
            rr = jnp.full((L,), r, jnp.int32)
            u = plsc.load_gather(idx, [jnp.full((L,), j, jnp.int32), rr])
            o_e = (u & 3) * D_GMF + iota
            o_m = (u & 1) * D_MLP + iota
            for c in range(D_MLP // L):
                vals = plsc.load_gather(mlp_g, [rr, o_m + c * L])
                pack_v[r, pl.ds(c * L, L)] = vals
            for c in range(D_GMF // L):
                vals = plsc.load_gather(emb_g, [rr, o_e + c * L])
                pack_v[r, pl.ds(D_MLP + c * L, L)] = vals
            return _
        lax.fori_loop(0, CHUNK, extract, 0)
        pltpu.sync_copy(pack_v, out.at[pl.ds(base + j * CHUNK, CHUNK)])


def _sc_gather_side(ids2d, emb_c, mlp_c):
    mesh = plsc.VectorSubcoreMesh(core_axis_name="c", subcore_axis_name="s")
    f = functools.partial(
        pl.kernel,
        mesh=mesh,
        out_type=jax.ShapeDtypeStruct((B, 128), jnp.float32),
        scratch_types=[
            pltpu.VMEM((CHUNKS_PER_W, CHUNK), jnp.int32),
            pltpu.VMEM((CHUNKS_PER_W, CHUNK), jnp.int32),
            pltpu.VMEM((CHUNKS_PER_W, CHUNK), jnp.int32),
            pltpu.VMEM((CHUNK, 128), jnp.float32),
            pltpu.VMEM((CHUNK, 128), jnp.float32),
            pltpu.VMEM((CHUNK, 128), jnp.float32),
            pltpu.SemaphoreType.DMA,
        ],
        compiler_params=pltpu.CompilerParams(
            use_tc_tiling_on_sc=True, needs_layout_passes=False),
    )(_gather_side_body)
    return f(ids2d, emb_c, mlp_c)


BLK = 2048


def _dense_body(u_ref, i_ref, W1_ref, b1_ref, W2_ref, b2_ref,
                Wo_ref, bo_ref, out_ref):
    u = u_ref[...]
    i = i_ref[...]
    gmf = (u[:, D_MLP:D_MLP + D_GMF] * i[:, D_MLP:D_MLP + D_GMF])
    x = jnp.dot(u[:, 0:D_MLP], W1_ref[0:D_MLP, :],
                preferred_element_type=jnp.float32)
    x = x + jnp.dot(i[:, 0:D_MLP], W1_ref[D_MLP:2 * D_MLP, :],
                    preferred_element_type=jnp.float32)
    x = jax.nn.relu(x + b1_ref[...])
    x = jnp.dot(x, W2_ref[...], preferred_element_type=jnp.float32)
    x = jax.nn.relu(x + b2_ref[...])
    wg = Wo_ref[0:D_GMF, 0]
    wm = Wo_ref[D_GMF:2 * D_GMF, 0]
    logit = (jnp.sum(gmf * wg[None, :], axis=-1)
             + jnp.sum(x * wm[None, :], axis=-1)
             + bo_ref[...])
    out_ref[...] = jax.nn.sigmoid(logit)


def _tc_dense(u_pack, i_pack, W1, b1, W2, b2, Wo, bo):
    grid = (B // BLK,)
    blk = pl.BlockSpec((BLK, 128), lambda i: (i, 0))
    full = lambda s: pl.BlockSpec(s, lambda i: tuple(0 for _ in s))
    return pl.pallas_call(
        _dense_body,
        grid=grid,
        in_specs=[
            blk, blk,
            full(W1.shape), full(b1.shape), full(W2.shape), full(b2.shape),
            full(Wo.shape), full(bo.shape),
        ],
        out_specs=pl.BlockSpec((BLK,), lambda i: (i,)),
        out_shape=jax.ShapeDtypeStruct((B,), jnp.float32),
    )(u_pack, i_pack, W1, b1, W2, b2, Wo, bo)


def kernel(user_emb, item_emb, user_emb_mlp, item_emb_mlp,
           W1, b1, W2, b2, Wo, bo, user_ids, movie_ids):
    uid2d = user_ids.astype(jnp.int32).reshape(B // CHUNK, CHUNK)
    mid2d = movie_ids.astype(jnp.int32).reshape(B // CHUNK, CHUNK)
    ue_c, ie_c, um_c, im_c = _tc_compact(
        user_emb, item_emb, user_emb_mlp, item_emb_mlp)
    u_pack = _sc_gather_side(uid2d, ue_c, um_c)
    i_pack = _sc_gather_side(mid2d, ie_c, im_c)
    return _tc_dense(u_pack, i_pack, W1, b1, W2, b2, Wo, bo)


# speedup vs baseline: 1.3789x; 1.3789x over previous
import functools

import jax
import jax.numpy as jnp
from jax import lax
from jax.experimental import pallas as pl
from jax.experimental.pallas import tpu as pltpu
from jax.experimental.pallas import tpu_sc as plsc

B = 16384
V = 100000
D_GMF = 32
D_MLP = 64
NC = 2
NS = 16
NW = NC * NS
ROWS_PER_W = B // NW
CHUNK = 128
CHUNKS_PER_W = ROWS_PER_W // CHUNK
VBLK = 4096
L = 16


def _compact_body(ue_ref, ie_ref, um_ref, im_ref,
                  ue_o, ie_o, um_o, im_o):
    def pack(xt, r):
        x = jnp.transpose(xt)
        n = x.shape[0] // r
        x3 = x.reshape(n, r, x.shape[1])
        return jnp.concatenate([x3[:, j, :] for j in range(r)], axis=-1)
    ue_o[...] = pack(ue_ref[...], 4)
    ie_o[...] = pack(ie_ref[...], 4)
    um_o[...] = pack(um_ref[...], 2)
    im_o[...] = pack(im_ref[...], 2)


def _tc_compact(ue_t, ie_t, um_t, im_t):
    grid = (pl.cdiv(V, VBLK),)
    bin32 = pl.BlockSpec((D_GMF, VBLK), lambda i: (0, i))
    bin64 = pl.BlockSpec((D_MLP, VBLK), lambda i: (0, i))
    bo32 = pl.BlockSpec((VBLK // 4, 128), lambda i: (i, 0))
    bo64 = pl.BlockSpec((VBLK // 2, 128), lambda i: (i, 0))
    return pl.pallas_call(
        _compact_body,
        grid=grid,
        in_specs=[bin32, bin32, bin64, bin64],
        out_specs=[bo32, bo32, bo64, bo64],
        out_shape=[
            jax.ShapeDtypeStruct((V // 4, 128), jnp.float32),
            jax.ShapeDtypeStruct((V // 4, 128), jnp.float32),
            jax.ShapeDtypeStruct((V // 2, 128), jnp.float32),
            jax.ShapeDtypeStruct((V // 2, 128), jnp.float32),
        ],
    )(ue_t, ie_t, um_t, im_t)


def _gather_side_body(ids_hbm, emb_hbm, mlp_hbm, out,
                      idx, ge_idx, gm_idx, emb_g, mlp_g, pack_v, sem):
    wid = lax.axis_index("s") * NC + lax.axis_index("c")
    base = wid * ROWS_PER_W
    pltpu.sync_copy(ids_hbm.at[pl.ds(wid * CHUNKS_PER_W, CHUNKS_PER_W)], idx)
    for j in range(CHUNKS_PER_W):
        for c in range(CHUNK // L):
            sl = pl.ds(c * L, L)
            v = idx[j, sl]
            ge_idx[j, sl] = v >> 2
            gm_idx[j, sl] = v >> 1
    iota = lax.iota(jnp.int32, L)
    for j in range(CHUNKS_PER_W):
        pltpu.async_copy(emb_hbm.at[ge_idx.at[j]], emb_g, sem).wait()
        pltpu.async_copy(mlp_hbm.at[gm_idx.at[j]], mlp_g, sem).wait()

        def extract(r, _):
            rr = jnp.full((L,), r, jnp.int32)
            u = plsc.load_gather(idx, [jnp.full((L,), j, jnp.int32), rr])
            o_e = (u & 3) * D_GMF + iota
            o_m = (u & 1) * D_MLP + iota
            for c in range(D_MLP // L):
                vals = plsc.load_gather(mlp_g, [rr, o_m + c * L])
                pack_v[r, pl.ds(c * L, L)] = vals
            for c in range(D_GMF // L):
                vals = plsc.load_gather(emb_g, [rr, o_e + c * L])
                pack_v[r, pl.ds(D_MLP + c * L, L)] = vals
            return _
        lax.fori_loop(0, CHUNK, extract, 0)
        pltpu.sync_copy(pack_v, out.at[pl.ds(base + j * CHUNK, CHUNK)])


def _sc_gather_side(ids2d, emb_c, mlp_c):
    mesh = plsc.VectorSubcoreMesh(core_axis_name="c", subcore_axis_name="s")
    f = functools.partial(
        pl.kernel,
        mesh=mesh,
        out_type=jax.ShapeDtypeStruct((B, 128), jnp.float32),
        scratch_types=[
            pltpu.VMEM((CHUNKS_PER_W, CHUNK), jnp.int32),
            pltpu.VMEM((CHUNKS_PER_W, CHUNK), jnp.int32),
            pltpu.VMEM((CHUNKS_PER_W, CHUNK), jnp.int32),
            pltpu.VMEM((CHUNK, 128), jnp.float32),
            pltpu.VMEM((CHUNK, 128), jnp.float32),
            pltpu.VMEM((CHUNK, 128), jnp.float32),
            pltpu.SemaphoreType.DMA,
        ],
        compiler_params=pltpu.CompilerParams(
            use_tc_tiling_on_sc=True, needs_layout_passes=False),
    )(_gather_side_body)
    return f(ids2d, emb_c, mlp_c)


BLK = 2048


def _dense_body(u_ref, i_ref, W1_ref, b1_ref, W2_ref, b2_ref,
                Wo_ref, bo_ref, out_ref):
    u = u_ref[...]
    i = i_ref[...]
    gmf = (u[:, D_MLP:D_MLP + D_GMF] * i[:, D_MLP:D_MLP + D_GMF])
    x = jnp.dot(u[:, 0:D_MLP], W1_ref[0:D_MLP, :],
                preferred_element_type=jnp.float32)
    x = x + jnp.dot(i[:, 0:D_MLP], W1_ref[D_MLP:2 * D_MLP, :],
                    preferred_element_type=jnp.float32)
    x = jax.nn.relu(x + b1_ref[...])
    x = jnp.dot(x, W2_ref[...], preferred_element_type=jnp.float32)
    x = jax.nn.relu(x + b2_ref[...])
    wg = Wo_ref[0:D_GMF, 0]
    wm = Wo_ref[D_GMF:2 * D_GMF, 0]
    logit = (jnp.sum(gmf * wg[None, :], axis=-1)
             + jnp.sum(x * wm[None, :], axis=-1)
             + bo_ref[...])
    out_ref[...] = jax.nn.sigmoid(logit)


def _tc_dense(u_pack, i_pack, W1, b1, W2, b2, Wo, bo):
    grid = (B // BLK,)
    blk = pl.BlockSpec((BLK, 128), lambda i: (i, 0))
    full = lambda s: pl.BlockSpec(s, lambda i: tuple(0 for _ in s))
    return pl.pallas_call(
        _dense_body,
        grid=grid,
        in_specs=[
            blk, blk,
            full(W1.shape), full(b1.shape), full(W2.shape), full(b2.shape),
            full(Wo.shape), full(bo.shape),
        ],
        out_specs=pl.BlockSpec((BLK,), lambda i: (i,)),
        out_shape=jax.ShapeDtypeStruct((B,), jnp.float32),
    )(u_pack, i_pack, W1, b1, W2, b2, Wo, bo)


def kernel(user_emb, item_emb, user_emb_mlp, item_emb_mlp,
           W1, b1, W2, b2, Wo, bo, user_ids, movie_ids):
    uid2d = user_ids.astype(jnp.int32).reshape(B // CHUNK, CHUNK)
    mid2d = movie_ids.astype(jnp.int32).reshape(B // CHUNK, CHUNK)
    ue_c, ie_c, um_c, im_c = _tc_compact(
        user_emb.T, item_emb.T, user_emb_mlp.T, item_emb_mlp.T)
    u_pack = _sc_gather_side(uid2d, ue_c, um_c)
    i_pack = _sc_gather_side(mid2d, ie_c, im_c)
    return _tc_dense(u_pack, i_pack, W1, b1, W2, b2, Wo, bo)


# speedup vs baseline: 1.4430x; 1.0465x over previous
import functools

import jax
import jax.numpy as jnp
from jax import lax
from jax.experimental import pallas as pl
from jax.experimental.pallas import tpu as pltpu
from jax.experimental.pallas import tpu_sc as plsc

B = 16384
V = 100000
D_GMF = 32
D_MLP = 64
NC = 2
NS = 16
NW = NC * NS
ROWS_PER_W = B // NW
CHUNK = 128
CHUNKS_PER_W = ROWS_PER_W // CHUNK
VBLK = 4096
L = 16


def _compact_body(emb_ref, mlp_ref, emb_o, mlp_o):
    def pack(xt, r):
        d = xt.shape[0]
        x = lax.dot_general(xt, jnp.eye(d, dtype=jnp.float32),
                            (((0,), (0,)), ((), ())),
                            preferred_element_type=jnp.float32)
        n = x.shape[0] // r
        x3 = x.reshape(n, r, d)
        return jnp.concatenate([x3[:, j, :] for j in range(r)], axis=-1)
    emb_o[...] = pack(emb_ref[...], 4)
    mlp_o[...] = pack(mlp_ref[...], 2)


def _tc_compact(emb_t, mlp_t):
    grid = (pl.cdiv(V, VBLK),)
    bin32 = pl.BlockSpec((D_GMF, VBLK), lambda i: (0, i))
    bin64 = pl.BlockSpec((D_MLP, VBLK), lambda i: (0, i))
    bo32 = pl.BlockSpec((VBLK // 4, 128), lambda i: (i, 0))
    bo64 = pl.BlockSpec((VBLK // 2, 128), lambda i: (i, 0))
    return pl.pallas_call(
        _compact_body,
        grid=grid,
        in_specs=[bin32, bin64],
        out_specs=[bo32, bo64],
        out_shape=[
            jax.ShapeDtypeStruct((V // 4, 128), jnp.float32),
            jax.ShapeDtypeStruct((V // 2, 128), jnp.float32),
        ],
    )(emb_t, mlp_t)


def _gather_side_body(ids_hbm, emb_hbm, mlp_hbm, out,
                      idx, ge_idx, gm_idx, emb_g, mlp_g, pack_v, sem):
    wid = lax.axis_index("s") * NC + lax.axis_index("c")
    base = wid * ROWS_PER_W
    pltpu.sync_copy(ids_hbm.at[pl.ds(wid * CHUNKS_PER_W, CHUNKS_PER_W)], idx)
    for j in range(CHUNKS_PER_W):
        for c in range(CHUNK // L):
            sl = pl.ds(c * L, L)
            v = idx[j, sl]
            ge_idx[j, sl] = v >> 2
            gm_idx[j, sl] = v >> 1
    iota = lax.iota(jnp.int32, L)
    for j in range(CHUNKS_PER_W):
        pltpu.async_copy(emb_hbm.at[ge_idx.at[j]], emb_g, sem).wait()
        pltpu.async_copy(mlp_hbm.at[gm_idx.at[j]], mlp_g, sem).wait()

        def extract(r, _):
            rr = jnp.full((L,), r, jnp.int32)
            u = plsc.load_gather(idx, [jnp.full((L,), j, jnp.int32), rr])
            o_e = (u & 3) * D_GMF + iota
            o_m = (u & 1) * D_MLP + iota
            for c in range(D_MLP // L):
                vals = plsc.load_gather(mlp_g, [rr, o_m + c * L])
                pack_v[r, pl.ds(c * L, L)] = vals
            for c in range(D_GMF // L):
                vals = plsc.load_gather(emb_g, [rr, o_e + c * L])
                pack_v[r, pl.ds(D_MLP + c * L, L)] = vals
            return _
        lax.fori_loop(0, CHUNK, extract, 0)
        pltpu.sync_copy(pack_v, out.at[pl.ds(base + j * CHUNK, CHUNK)])


def _sc_gather_side(ids2d, emb_c, mlp_c):
    mesh = plsc.VectorSubcoreMesh(core_axis_name="c", subcore_axis_name="s")
    f = functools.partial(
        pl.kernel,
        mesh=mesh,
        out_type=jax.ShapeDtypeStruct((B, 128), jnp.float32),
        scratch_types=[
            pltpu.VMEM((CHUNKS_PER_W, CHUNK), jnp.int32),
            pltpu.VMEM((CHUNKS_PER_W, CHUNK), jnp.int32),
            pltpu.VMEM((CHUNKS_PER_W, CHUNK), jnp.int32),
            pltpu.VMEM((CHUNK, 128), jnp.float32),
            pltpu.VMEM((CHUNK, 128), jnp.float32),
            pltpu.VMEM((CHUNK, 128), jnp.float32),
            pltpu.SemaphoreType.DMA,
        ],
        compiler_params=pltpu.CompilerParams(
            use_tc_tiling_on_sc=True, needs_layout_passes=False),
    )(_gather_side_body)
    return f(ids2d, emb_c, mlp_c)


BLK = 2048


def _dense_body(u_ref, i_ref, W1_ref, b1_ref, W2_ref, b2_ref,
                Wo_ref, bo_ref, out_ref):
    u = u_ref[...]
    i = i_ref[...]
    gmf = (u[:, D_MLP:D_MLP + D_GMF] * i[:, D_MLP:D_MLP + D_GMF])
    x = jnp.dot(u[:, 0:D_MLP], W1_ref[0:D_MLP, :],
                preferred_element_type=jnp.float32)
    x = x + jnp.dot(i[:, 0:D_MLP], W1_ref[D_MLP:2 * D_MLP, :],
                    preferred_element_type=jnp.float32)
    x = jax.nn.relu(x + b1_ref[...])
    x = jnp.dot(x, W2_ref[...], preferred_element_type=jnp.float32)
    x = jax.nn.relu(x + b2_ref[...])
    wg = Wo_ref[0:D_GMF, 0]
    wm = Wo_ref[D_GMF:2 * D_GMF, 0]
    logit = (jnp.sum(gmf * wg[None, :], axis=-1)
             + jnp.sum(x * wm[None, :], axis=-1)
             + bo_ref[...])
    out_ref[...] = jax.nn.sigmoid(logit)


def _tc_dense(u_pack, i_pack, W1, b1, W2, b2, Wo, bo):
    grid = (B // BLK,)
    blk = pl.BlockSpec((BLK, 128), lambda i: (i, 0))
    full = lambda s: pl.BlockSpec(s, lambda i: tuple(0 for _ in s))
    return pl.pallas_call(
        _dense_body,
        grid=grid,
        in_specs=[
            blk, blk,
            full(W1.shape), full(b1.shape), full(W2.shape), full(b2.shape),
            full(Wo.shape), full(bo.shape),
        ],
        out_specs=pl.BlockSpec((BLK,), lambda i: (i,)),
        out_shape=jax.ShapeDtypeStruct((B,), jnp.float32),
    )(u_pack, i_pack, W1, b1, W2, b2, Wo, bo)


def kernel(user_emb, item_emb, user_emb_mlp, item_emb_mlp,
           W1, b1, W2, b2, Wo, bo, user_ids, movie_ids):
    uid2d = user_ids.astype(jnp.int32).reshape(B // CHUNK, CHUNK)
    mid2d = movie_ids.astype(jnp.int32).reshape(B // CHUNK, CHUNK)
    ue_c, um_c = _tc_compact(user_emb.T, user_emb_mlp.T)
    u_pack = _sc_gather_side(uid2d, ue_c, um_c)
    ie_c, im_c = _tc_compact(item_emb.T, item_emb_mlp.T)
    i_pack = _sc_gather_side(mid2d, ie_c, im_c)
    return _tc_dense(u_pack, i_pack, W1, b1, W2, b2, Wo, bo)


# speedup vs baseline: 2.4112x; 1.6709x over previous
import functools

import jax
import jax.numpy as jnp
from jax import lax
from jax.experimental import pallas as pl
from jax.experimental.pallas import tpu as pltpu
from jax.experimental.pallas import tpu_sc as plsc

B = 16384
V = 100000
D_GMF = 32
D_MLP = 64
NC = 2
NS = 16
NW = NC * NS
ROWS_PER_W = B // NW
CHUNK = 128
CHUNKS_PER_W = ROWS_PER_W // CHUNK
VBLK = 4096
L = 16


def _compact_body(emb_ref, mlp_ref, out_ref):
    mlp = jnp.transpose(mlp_ref[...])
    emb = jnp.transpose(emb_ref[...])
    pad = jnp.zeros((emb.shape[0], 128 - D_MLP - D_GMF), jnp.float32)
    out_ref[...] = jnp.concatenate([mlp, emb, pad], axis=-1)


def _tc_compact(emb_t, mlp_t):
    grid = (pl.cdiv(V, VBLK),)
    return pl.pallas_call(
        _compact_body,
        grid=grid,
        in_specs=[
            pl.BlockSpec((D_GMF, VBLK), lambda i: (0, i)),
            pl.BlockSpec((D_MLP, VBLK), lambda i: (0, i)),
        ],
        out_specs=pl.BlockSpec((VBLK, 128), lambda i: (i, 0)),
        out_shape=jax.ShapeDtypeStruct((V, 128), jnp.float32),
    )(emb_t, mlp_t)


def _gather_side_body(ids_hbm, tbl_hbm, out, idx, g0, g1, sem):
    wid = lax.axis_index("s") * NC + lax.axis_index("c")
    base = wid * ROWS_PER_W
    pltpu.sync_copy(ids_hbm.at[pl.ds(wid * CHUNKS_PER_W, CHUNKS_PER_W)], idx)
    bufs = (g0, g1)
    copies = [None, None]
    for j in range(CHUNKS_PER_W):
        copies[j % 2] = pltpu.async_copy(
            tbl_hbm.at[idx.at[j]], bufs[j % 2], sem)
        if j >= 1:
            copies[(j - 1) % 2].wait()
            out_sl = pl.ds(base + (j - 1) * CHUNK, CHUNK)
            pltpu.sync_copy(bufs[(j - 1) % 2], out.at[out_sl])
    copies[(CHUNKS_PER_W - 1) % 2].wait()
    out_sl = pl.ds(base + (CHUNKS_PER_W - 1) * CHUNK, CHUNK)
    pltpu.sync_copy(bufs[(CHUNKS_PER_W - 1) % 2], out.at[out_sl])


def _sc_gather_side(ids2d, tbl_c):
    mesh = plsc.VectorSubcoreMesh(core_axis_name="c", subcore_axis_name="s")
    f = functools.partial(
        pl.kernel,
        mesh=mesh,
        out_type=jax.ShapeDtypeStruct((B, 128), jnp.float32),
        scratch_types=[
            pltpu.VMEM((CHUNKS_PER_W, CHUNK), jnp.int32),
            pltpu.VMEM((CHUNK, 128), jnp.float32),
            pltpu.VMEM((CHUNK, 128), jnp.float32),
            pltpu.SemaphoreType.DMA,
        ],
        compiler_params=pltpu.CompilerParams(
            use_tc_tiling_on_sc=True, needs_layout_passes=False),
    )(_gather_side_body)
    return f(ids2d, tbl_c)


BLK = 2048


def _dense_body(u_ref, i_ref, W1_ref, b1_ref, W2_ref, b2_ref,
                Wo_ref, bo_ref, out_ref):
    u = u_ref[...]
    i = i_ref[...]
    gmf = (u[:, D_MLP:D_MLP + D_GMF] * i[:, D_MLP:D_MLP + D_GMF])
    x = jnp.dot(u[:, 0:D_MLP], W1_ref[0:D_MLP, :],
                preferred_element_type=jnp.float32)
    x = x + jnp.dot(i[:, 0:D_MLP], W1_ref[D_MLP:2 * D_MLP, :],
                    preferred_element_type=jnp.float32)
    x = jax.nn.relu(x + b1_ref[...])
    x = jnp.dot(x, W2_ref[...], preferred_element_type=jnp.float32)
    x = jax.nn.relu(x + b2_ref[...])
    wg = Wo_ref[0:D_GMF, 0]
    wm = Wo_ref[D_GMF:2 * D_GMF, 0]
    logit = (jnp.sum(gmf * wg[None, :], axis=-1)
             + jnp.sum(x * wm[None, :], axis=-1)
             + bo_ref[...])
    out_ref[...] = jax.nn.sigmoid(logit)


def _tc_dense(u_pack, i_pack, W1, b1, W2, b2, Wo, bo):
    grid = (B // BLK,)
    blk = pl.BlockSpec((BLK, 128), lambda i: (i, 0))
    full = lambda s: pl.BlockSpec(s, lambda i: tuple(0 for _ in s))
    return pl.pallas_call(
        _dense_body,
        grid=grid,
        in_specs=[
            blk, blk,
            full(W1.shape), full(b1.shape), full(W2.shape), full(b2.shape),
            full(Wo.shape), full(bo.shape),
        ],
        out_specs=pl.BlockSpec((BLK,), lambda i: (i,)),
        out_shape=jax.ShapeDtypeStruct((B,), jnp.float32),
    )(u_pack, i_pack, W1, b1, W2, b2, Wo, bo)


def kernel(user_emb, item_emb, user_emb_mlp, item_emb_mlp,
           W1, b1, W2, b2, Wo, bo, user_ids, movie_ids):
    uid2d = user_ids.astype(jnp.int32).reshape(B // CHUNK, CHUNK)
    mid2d = movie_ids.astype(jnp.int32).reshape(B // CHUNK, CHUNK)
    u_tbl = _tc_compact(user_emb.T, user_emb_mlp.T)
    u_pack = _sc_gather_side(uid2d, u_tbl)
    i_tbl = _tc_compact(item_emb.T, item_emb_mlp.T)
    i_pack = _sc_gather_side(mid2d, i_tbl)
    return _tc_dense(u_pack, i_pack, W1, b1, W2, b2, Wo, bo)


# speedup vs baseline: 2.5933x; 1.0755x over previous
import functools

import jax
import jax.numpy as jnp
from jax import lax
from jax.experimental import pallas as pl
from jax.experimental.pallas import tpu as pltpu
from jax.experimental.pallas import tpu_sc as plsc

B = 16384
V = 100000
D_GMF = 32
D_MLP = 64
NC = 2
NS = 16
NW = NC * NS
ROWS_PER_W = B // NW
CHUNK = 128
CHUNKS_PER_W = ROWS_PER_W // CHUNK
VBLK = 8192
L = 16


def _compact_body(emb_ref, mlp_ref, out_ref):
    mlp = jnp.transpose(mlp_ref[...])
    emb = jnp.transpose(emb_ref[...])
    pad = jnp.zeros((emb.shape[0], 128 - D_MLP - D_GMF), jnp.float32)
    out_ref[...] = jnp.concatenate([mlp, emb, pad], axis=-1)


def _tc_compact(emb_t, mlp_t):
    grid = (pl.cdiv(V, VBLK),)
    return pl.pallas_call(
        _compact_body,
        grid=grid,
        in_specs=[
            pl.BlockSpec((D_GMF, VBLK), lambda i: (0, i)),
            pl.BlockSpec((D_MLP, VBLK), lambda i: (0, i)),
        ],
        out_specs=pl.BlockSpec((VBLK, 128), lambda i: (i, 0)),
        out_shape=jax.ShapeDtypeStruct((V, 128), jnp.float32),
    )(emb_t, mlp_t)


def _gather_side_body(ids_hbm, tbl_hbm, out, idx, g0, g1, sem):
    wid = lax.axis_index("s") * NC + lax.axis_index("c")
    base = wid * ROWS_PER_W
    pltpu.sync_copy(ids_hbm.at[pl.ds(wid * CHUNKS_PER_W, CHUNKS_PER_W)], idx)
    bufs = (g0, g1)
    copies = [None, None]
    for j in range(CHUNKS_PER_W):
        copies[j % 2] = pltpu.async_copy(
            tbl_hbm.at[idx.at[j]], bufs[j % 2], sem)
        if j >= 1:
            copies[(j - 1) % 2].wait()
            out_sl = pl.ds(base + (j - 1) * CHUNK, CHUNK)
            pltpu.sync_copy(bufs[(j - 1) % 2], out.at[out_sl])
    copies[(CHUNKS_PER_W - 1) % 2].wait()
    out_sl = pl.ds(base + (CHUNKS_PER_W - 1) * CHUNK, CHUNK)
    pltpu.sync_copy(bufs[(CHUNKS_PER_W - 1) % 2], out.at[out_sl])


def _sc_gather_side(ids2d, tbl_c):
    mesh = plsc.VectorSubcoreMesh(core_axis_name="c", subcore_axis_name="s")
    f = functools.partial(
        pl.kernel,
        mesh=mesh,
        out_type=jax.ShapeDtypeStruct((B, 128), jnp.float32),
        scratch_types=[
            pltpu.VMEM((CHUNKS_PER_W, CHUNK), jnp.int32),
            pltpu.VMEM((CHUNK, 128), jnp.float32),
            pltpu.VMEM((CHUNK, 128), jnp.float32),
            pltpu.SemaphoreType.DMA,
        ],
        compiler_params=pltpu.CompilerParams(
            use_tc_tiling_on_sc=True, needs_layout_passes=False),
    )(_gather_side_body)
    return f(ids2d, tbl_c)


BLK = 4096


def _dense_body(u_ref, i_ref, W1_ref, b1_ref, W2_ref, b2_ref,
                Wo_ref, bo_ref, out_ref):
    u = u_ref[...]
    i = i_ref[...]
    gmf = (u[:, D_MLP:D_MLP + D_GMF] * i[:, D_MLP:D_MLP + D_GMF])
    x = jnp.dot(u[:, 0:D_MLP], W1_ref[0:D_MLP, :],
                preferred_element_type=jnp.float32)
    x = x + jnp.dot(i[:, 0:D_MLP], W1_ref[D_MLP:2 * D_MLP, :],
                    preferred_element_type=jnp.float32)
    x = jax.nn.relu(x + b1_ref[...])
    x = jnp.dot(x, W2_ref[...], preferred_element_type=jnp.float32)
    x = jax.nn.relu(x + b2_ref[...])
    wg = Wo_ref[0:D_GMF, 0]
    wm = Wo_ref[D_GMF:2 * D_GMF, 0]
    logit = (jnp.sum(gmf * wg[None, :], axis=-1)
             + jnp.sum(x * wm[None, :], axis=-1)
             + bo_ref[...])
    out_ref[...] = jax.nn.sigmoid(logit)


def _tc_dense(u_pack, i_pack, W1, b1, W2, b2, Wo, bo):
    grid = (B // BLK,)
    blk = pl.BlockSpec((BLK, 128), lambda i: (i, 0))
    full = lambda s: pl.BlockSpec(s, lambda i: tuple(0 for _ in s))
    return pl.pallas_call(
        _dense_body,
        grid=grid,
        in_specs=[
            blk, blk,
            full(W1.shape), full(b1.shape), full(W2.shape), full(b2.shape),
            full(Wo.shape), full(bo.shape),
        ],
        out_specs=pl.BlockSpec((BLK,), lambda i: (i,)),
        out_shape=jax.ShapeDtypeStruct((B,), jnp.float32),
    )(u_pack, i_pack, W1, b1, W2, b2, Wo, bo)


def kernel(user_emb, item_emb, user_emb_mlp, item_emb_mlp,
           W1, b1, W2, b2, Wo, bo, user_ids, movie_ids):
    uid2d = user_ids.astype(jnp.int32).reshape(B // CHUNK, CHUNK)
    mid2d = movie_ids.astype(jnp.int32).reshape(B // CHUNK, CHUNK)
    u_tbl = _tc_compact(user_emb.T, user_emb_mlp.T)
    u_pack = _sc_gather_side(uid2d, u_tbl)
    i_tbl = _tc_compact(item_emb.T, item_emb_mlp.T)
    i_pack = _sc_gather_side(mid2d, i_tbl)
    return _tc_dense(u_pack, i_pack, W1, b1, W2, b2, Wo, bo)
